# baseline (speedup 1.0000x reference)
import functools

import jax
import jax.numpy as jnp
from jax import lax
from jax.experimental import pallas as pl
from jax.experimental.pallas import tpu as pltpu
from jax.experimental.pallas import tpu_sc as plsc

N = 10000
E = 320000
D = 128
R = 1000
B = 4096
NG = 2

NC = 2
NS = 16
NW = NC * NS
L = 16

N_PAD = 10240
DH = D // NC
EP = (E + NW - 1) // NW
EP = ((EP + 127) // 128) * 128
CH = EP // 128
EP2 = 2 * EP
KE = 128
CHB = EP2 // KE
ROWS_PER_TILE = N_PAD // NS
BPW = B // NW


@functools.cache
def _mesh():
    return plsc.VectorSubcoreMesh(
        core_axis_name="c", subcore_axis_name="s",
        num_cores=NC, num_subcores=NS)


def _worker_id():
    return lax.axis_index("s") * NC + lax.axis_index("c")


def _zero_vmem(ref, rows, cols):
    z = jnp.zeros((L,), jnp.float32)

    def row(i, carry):
        for k in range(cols // L):
            ref[i, pl.ds(k * L, L)] = z
        return carry

    lax.fori_loop(0, rows, row, 0, unroll=False)


def _deg_body(didx_hbm, deg_out, idx_v, deg_v):
    wid = _worker_id()
    one = jnp.ones((L,), jnp.float32)

    def zero_deg(i, carry):
        deg_v[i, :] = jnp.zeros((L,), jnp.float32)
        return carry

    for gi in range(NG):
        lax.fori_loop(0, N_PAD // L, zero_deg, 0, unroll=False)
        pltpu.sync_copy(didx_hbm.at[gi, wid], idx_v)

        def chunk(j, carry):
            for k in range(128 // L):
                idx = idx_v[j, pl.ds(k * L, L)]
                plsc.addupdate_scatter(deg_v, [idx >> 4, idx & 15], one)
            return carry

        lax.fori_loop(0, CH, chunk, 0, unroll=False)
        pltpu.sync_copy(deg_v, deg_out.at[gi, wid])


@functools.cache
def _deg_kernel():
  return pl.kernel(
    _deg_body,
    out_type=jax.ShapeDtypeStruct((NG, NW, N_PAD // L, L), jnp.float32),
    mesh=_mesh(),
    scratch_types=[
        pltpu.VMEM((CH, 128), jnp.int32),
        pltpu.VMEM((N_PAD // L, L), jnp.float32),
    ],
    compiler_params=pltpu.CompilerParams(
        needs_layout_passes=False, use_tc_tiling_on_sc=False),
  )


_NB = 2


def _agg_body(h_hbm, sidx_hbm, didx_hbm, acc_out, *scr):
    sidx_v, didx_v = scr[0], scr[1]
    rows = scr[2:2 + _NB]
    acc_sh = scr[2 + _NB]
    gsem = scr[3 + _NB:3 + 2 * _NB]
    ssem0 = scr[3 + 2 * _NB]
    cid = lax.axis_index("c")
    sid = lax.axis_index("s")
    base = sid * ROWS_PER_TILE
    bufA, bufB = rows[0], rows[1]
    ga, gb = gsem[0], gsem[1]

    def run(h_view):
        def g(j, buf, sem):
            pltpu.async_copy(h_view.at[sidx_v.at[j]], buf, sem)

        def gw(j, buf, sem):
            pltpu.make_async_copy(h_view.at[sidx_v.at[j]], buf, sem).wait()

        def s(j, buf):
            pltpu.async_copy(buf, acc_sh.at[didx_v.at[j]], ssem0, add=True)

        def sw(j, buf):
            pltpu.make_async_copy(buf, acc_sh.at[didx_v.at[j]], ssem0).wait()

        g(0, bufA, ga)
        g(1, bufB, gb)
        gw(0, bufA, ga)
        s(0, bufA)
        g(2, bufA, ga)

        def group(k, carry):
            j1 = 2 * k + 1
            j2 = j1 + 1
            gw(j1, bufB, gb)
            sw(j1 - 1, bufA)
            s(j1, bufB)
            g(j1 + 2, bufB, gb)
            gw(j2, bufA, ga)
            sw(j1, bufB)
            s(j2, bufA)

            @pl.when(j2 + 2 < CHB)
            def _():
                g(j2 + 2, bufA, ga)

            return carry

        lax.fori_loop(0, (CHB - 2) // 2, group, 0, unroll=False)
        j = CHB - 1
        gw(j, bufB, gb)
        sw(j - 1, bufA)
        s(j, bufB)
        sw(j, bufB)

    for gi in range(NG):
        _zero_vmem(bufA, KE, DH)
        for t in range(ROWS_PER_TILE // KE):
            pltpu.sync_copy(bufA, acc_sh.at[pl.ds(base + t * KE, KE)])
        plsc.subcore_barrier()

        pltpu.sync_copy(sidx_hbm.at[gi, sid], sidx_v)
        pltpu.sync_copy(didx_hbm.at[gi, sid], didx_v)

        @pl.when(cid == 0)
        def _():
            run(h_hbm.at[gi, 0])

        @pl.when(cid == 1)
        def _():
            run(h_hbm.at[gi, 1])

        plsc.subcore_barrier()
        pltpu.sync_copy(acc_sh.at[pl.ds(base, ROWS_PER_TILE)],
                        acc_out.at[gi, cid, pl.ds(base, ROWS_PER_TILE)])


@functools.cache
def _agg_kernel():
  return pl.kernel(
    _agg_body,
    out_type=jax.ShapeDtypeStruct((NG, NC, N_PAD, DH), jnp.float32),
    mesh=_mesh(),
    scratch_types=[
        pltpu.VMEM((CHB, KE), jnp.int32),
        pltpu.VMEM((CHB, KE), jnp.int32),
    ] + [pltpu.VMEM((KE, DH), jnp.float32)] * _NB + [
        pltpu.VMEM_SHARED((N_PAD, DH), jnp.float32),
    ] + [pltpu.SemaphoreType.DMA] * (_NB + 1),
    compiler_params=pltpu.CompilerParams(use_tc_tiling_on_sc=False),
  )


def _lookup_body(ent_hbm, rel_hbm, eidx_hbm, ridx_hbm,
                 esr_out, etg_out, rsr_out, rtg_out,
                 idx_v, rows_v, sem):
    wid = _worker_id()
    for table, idx_hbm, t, out in ((ent_hbm, eidx_hbm, 0, esr_out),
                                   (ent_hbm, eidx_hbm, 1, etg_out),
                                   (rel_hbm, ridx_hbm, 0, rsr_out),
                                   (rel_hbm, ridx_hbm, 1, rtg_out)):
        pltpu.sync_copy(idx_hbm.at[t, wid], idx_v)
        pltpu.async_copy(table.at[idx_v], rows_v, sem).wait()
        pltpu.sync_copy(rows_v, out.at[pl.ds(wid * BPW, BPW)])


@functools.cache
def _lookup_kernel():
  return pl.kernel(
    _lookup_body,
    out_type=(jax.ShapeDtypeStruct((B, D), jnp.float32),) * 4,
    mesh=_mesh(),
    scratch_types=[
        pltpu.VMEM((BPW,), jnp.int32),
        pltpu.VMEM((BPW, D), jnp.float32),
        pltpu.SemaphoreType.DMA,
    ],
    compiler_params=pltpu.CompilerParams(use_tc_tiling_on_sc=False),
  )


_RB = 1024


def _dinv_block(dp_ref):
    deg = jnp.sum(dp_ref[0], axis=0)[:, None] + 1.0
    return lax.rsqrt(deg)


def _store_split(o_ref, res):
    o_ref[0, 0] = res[:, :DH]
    o_ref[0, 1] = res[:, DH:]


def _mm_body(x_ref, w_ref, dp_ref, o_ref):
    dinv = _dinv_block(dp_ref)
    res = jnp.dot(x_ref[0], w_ref[...],
                  preferred_element_type=jnp.float32) * dinv
    _store_split(o_ref, res)


def _relu_halves(a_ref, h_ref, dinv):
    g0 = jnp.maximum((a_ref[0, 0] + h_ref[0, 0]) * dinv, 0.0)
    g1 = jnp.maximum((a_ref[0, 1] + h_ref[0, 1]) * dinv, 0.0)
    return g0, g1


def _mid_body(a_ref, h_ref, w_ref, dp_ref, o_ref):
    dinv = _dinv_block(dp_ref)
    g0, g1 = _relu_halves(a_ref, h_ref, dinv)
    res = (jnp.dot(g0, w_ref[:DH, :], preferred_element_type=jnp.float32)
           + jnp.dot(g1, w_ref[DH:, :], preferred_element_type=jnp.float32)
           ) * dinv
    _store_split(o_ref, res)


def _fin_body(a_ref, h_ref, dp_ref, o_ref):
    dinv = _dinv_block(dp_ref)
    g0, g1 = _relu_halves(a_ref, h_ref, dinv)
    o_ref[0, :, :DH] = g0
    o_ref[0, :, DH:] = g1


_row_spec = pl.BlockSpec((1, _RB, D), lambda g, i: (g, i, 0))
_split_spec = pl.BlockSpec((1, NC, _RB, DH), lambda g, i: (g, 0, i, 0))
_w_spec = pl.BlockSpec((D, D), lambda g, i: (0, 0))
_deg_spec = pl.BlockSpec((1, NW, _RB), lambda g, i: (g, 0, i))
_full_struct = jax.ShapeDtypeStruct((NG, N_PAD, D), jnp.float32)
_split_struct = jax.ShapeDtypeStruct((NG, NC, N_PAD, DH), jnp.float32)
_grid = (NG, N_PAD // _RB)

_tc_mm = pl.pallas_call(
    _mm_body, grid=_grid,
    in_specs=[_row_spec, _w_spec, _deg_spec],
    out_specs=_split_spec, out_shape=_split_struct)

_tc_mid = pl.pallas_call(
    _mid_body, grid=_grid,
    in_specs=[_split_spec, _split_spec, _w_spec, _deg_spec],
    out_specs=_split_spec, out_shape=_split_struct)

_tc_fin = pl.pallas_call(
    _fin_body, grid=_grid,
    in_specs=[_split_spec, _split_spec, _deg_spec],
    out_specs=_row_spec, out_shape=_full_struct)


def _prep_edges(edge_index):
    pad = NW * EP - E
    fill = jnp.full((pad,), N, jnp.int32)
    s = jnp.concatenate([edge_index[0], fill])
    d = jnp.concatenate([edge_index[1], fill])
    return (s.reshape(NS, CHB, KE), d.reshape(NS, CHB, KE),
            d.reshape(NW, CH, 128))


def kernel(entity_emb_sr, entity_emb_tg, rel_emb_sr, rel_emb_tg, W0, W1,
           edge_index_sr, edge_index_tg, sr_data, tg_data,
           sr_rel_data, tg_rel_data):
    s_sr, d_sr, d32_sr = _prep_edges(edge_index_sr)
    s_tg, d_tg, d32_tg = _prep_edges(edge_index_tg)
    sidx_all = jnp.stack([s_sr, s_tg])
    didx_all = jnp.stack([d_sr, d_tg])
    d32_all = jnp.stack([d32_sr, d32_tg])

    deg_all = _deg_kernel()(d32_all).reshape(NG, NW, N_PAD)

    pad_rows = jnp.zeros((N_PAD - N, D), jnp.float32)
    x_all = jnp.concatenate(
        [entity_emb_sr, pad_rows, entity_emb_tg, pad_rows]
    ).reshape(NG, N_PAD, D)

    h_all = _tc_mm(x_all, W0, deg_all)
    acc_all = _agg_kernel()(h_all, sidx_all, didx_all)
    h2_all = _tc_mid(acc_all, h_all, W1, deg_all)
    acc2_all = _agg_kernel()(h2_all, sidx_all, didx_all)
    g_all = _tc_fin(acc2_all, h2_all, deg_all)

    ent_table = g_all.reshape(NG * N_PAD, D)
    rel_table = jnp.concatenate([rel_emb_sr, rel_emb_tg])
    eidx = jnp.stack([sr_data, tg_data + N_PAD]).reshape(NG, NW, BPW)
    ridx = jnp.stack([sr_rel_data, tg_rel_data + R]).reshape(NG, NW, BPW)

    return _lookup_kernel()(ent_table, rel_table, eidx, ridx)


# speedup vs baseline: 1.1326x; 1.1326x over previous
import functools

import jax
import jax.numpy as jnp
from jax import lax
from jax.experimental import pallas as pl
from jax.experimental.pallas import tpu as pltpu
from jax.experimental.pallas import tpu_sc as plsc

N = 10000
E = 320000
D = 128
R = 1000
B = 4096

NC = 2
NS = 16
NW = NC * NS
L = 16

N_PAD = 10240
DH = D // NC
EP = (E + NW - 1) // NW
EP = ((EP + 127) // 128) * 128
CH = EP // 128
EP2 = 2 * EP
ROWS_PER_TILE = N_PAD // NS

@functools.cache
def _mesh():
    return plsc.VectorSubcoreMesh(
        core_axis_name="c", subcore_axis_name="s",
        num_cores=NC, num_subcores=NS)


def _worker_id():
    return lax.axis_index("s") * NC + lax.axis_index("c")


def _zero_vmem(ref, rows, cols):
    z = jnp.zeros((L,), jnp.float32)

    def row(i, carry):
        for k in range(cols // L):
            ref[i, pl.ds(k * L, L)] = z
        return carry

    lax.fori_loop(0, rows, row, 0, unroll=False)


def _deg_body(dsr_hbm, dtg_hbm, deg_sr_out, deg_tg_out, idx_v, deg_v):
    wid = _worker_id()
    one = jnp.ones((L,), jnp.float32)

    def zero_deg(i, carry):
        deg_v[i, :] = jnp.zeros((L,), jnp.float32)
        return carry

    for idx_hbm, out in ((dsr_hbm, deg_sr_out), (dtg_hbm, deg_tg_out)):
        lax.fori_loop(0, N_PAD // L, zero_deg, 0, unroll=False)
        pltpu.sync_copy(idx_hbm.at[wid], idx_v)

        def chunk(j, carry):
            for k in range(128 // L):
                idx = idx_v[j, pl.ds(k * L, L)]
                plsc.addupdate_scatter(deg_v, [idx >> 4, idx & 15], one)
            return carry

        lax.fori_loop(0, CH, chunk, 0, unroll=False)
        pltpu.sync_copy(deg_v, out.at[wid])


@functools.cache
def _deg_kernel():
  return pl.kernel(
    _deg_body,
    out_type=(jax.ShapeDtypeStruct((NW, N_PAD // L, L), jnp.float32),
              jax.ShapeDtypeStruct((NW, N_PAD // L, L), jnp.float32)),
    mesh=_mesh(),
    scratch_types=[
        pltpu.VMEM((CH, 128), jnp.int32),
        pltpu.VMEM((N_PAD // L, L), jnp.float32),
    ],
    compiler_params=pltpu.CompilerParams(
        needs_layout_passes=False, use_tc_tiling_on_sc=False),
  )


_NB = 2
KE = 128
CHB = EP2 // KE


def _agg_body(h0_hbm, h1_hbm, sidx_hbm, didx_hbm, acc_out, *scr):
    sidx_v, didx_v = scr[0], scr[1]
    rows = scr[2:2 + _NB]
    acc_sh = scr[2 + _NB]
    gsem = scr[3 + _NB:3 + 2 * _NB]
    ssem0 = scr[3 + 2 * _NB]
    cid = lax.axis_index("c")
    sid = lax.axis_index("s")
    base = sid * ROWS_PER_TILE
    r0 = rows[0]

    _zero_vmem(r0, KE, DH)
    for t in range(ROWS_PER_TILE // KE):
        pltpu.sync_copy(r0, acc_sh.at[pl.ds(base + t * KE, KE)])
    _REM = ROWS_PER_TILE % KE
    if _REM:
        pltpu.sync_copy(
            r0.at[pl.ds(0, _REM)],
            acc_sh.at[pl.ds(base + ROWS_PER_TILE - _REM, _REM)])
    plsc.subcore_barrier()

    pltpu.sync_copy(sidx_hbm.at[sid], sidx_v)
    pltpu.sync_copy(didx_hbm.at[sid], didx_v)

    def run(h_hbm):
        bufA, bufB = rows[0], rows[1]
        ga, gb = gsem[0], gsem[1]
        ss = ssem0

        def g(j, buf, sem):
            pltpu.async_copy(h_hbm.at[sidx_v.at[j]], buf, sem)

        def gw(j, buf, sem):
            pltpu.make_async_copy(h_hbm.at[sidx_v.at[j]], buf, sem).wait()

        def s(j, buf):
            pltpu.async_copy(buf, acc_sh.at[didx_v.at[j]], ss, add=True)

        def sw(j, buf):
            pltpu.make_async_copy(buf, acc_sh.at[didx_v.at[j]], ss).wait()

        g(0, bufA, ga)
        g(1, bufB, gb)
        gw(0, bufA, ga)
        s(0, bufA)
        g(2, bufA, ga)

        def group(k, carry):
            j1 = 2 * k + 1
            j2 = j1 + 1
            gw(j1, bufB, gb)
            sw(j1 - 1, bufA)
            s(j1, bufB)
            g(j1 + 2, bufB, gb)
            gw(j2, bufA, ga)
            sw(j1, bufB)
            s(j2, bufA)

            @pl.when(j2 + 2 < CHB)
            def _():
                g(j2 + 2, bufA, ga)

            return carry

        lax.fori_loop(0, (CHB - 2) // 2, group, 0, unroll=False)
        j = CHB - 1
        gw(j, bufB, gb)
        sw(j - 1, bufA)
        s(j, bufB)
        sw(j, bufB)

    @pl.when(cid == 0)
    def _():
        run(h0_hbm)

    @pl.when(cid == 1)
    def _():
        run(h1_hbm)

    plsc.subcore_barrier()
    pltpu.sync_copy(acc_sh.at[pl.ds(base, ROWS_PER_TILE)],
                    acc_out.at[cid, pl.ds(base, ROWS_PER_TILE)])


@functools.cache
def _agg_kernel():
  return pl.kernel(
    _agg_body,
    out_type=jax.ShapeDtypeStruct((NC, N_PAD, DH), jnp.float32),
    mesh=_mesh(),
    scratch_types=[
        pltpu.VMEM((CHB, KE), jnp.int32),
        pltpu.VMEM((CHB, KE), jnp.int32),
    ] + [pltpu.VMEM((KE, DH), jnp.float32)] * _NB + [
        pltpu.VMEM_SHARED((N_PAD, DH), jnp.float32),
    ] + [pltpu.SemaphoreType.DMA] * (_NB + 1),
    compiler_params=pltpu.CompilerParams(use_tc_tiling_on_sc=False),
  )


def _lookup_body(gsr_hbm, gtg_hbm, rsr_hbm, rtg_hbm,
                 isr_hbm, itg_hbm, irsr_hbm, irtg_hbm,
                 esr_out, etg_out, rsr_out, rtg_out,
                 idx_v, rows_v, sem):
    wid = _worker_id()
    bpw = B // NW
    for table, idx_hbm, out in ((gsr_hbm, isr_hbm, esr_out),
                                (gtg_hbm, itg_hbm, etg_out),
                                (rsr_hbm, irsr_hbm, rsr_out),
                                (rtg_hbm, irtg_hbm, rtg_out)):
        pltpu.sync_copy(idx_hbm.at[wid], idx_v)
        pltpu.async_copy(table.at[idx_v], rows_v, sem).wait()
        pltpu.sync_copy(rows_v, out.at[pl.ds(wid * bpw, bpw)])


@functools.cache
def _lookup_kernel():
  return pl.kernel(
    _lookup_body,
    out_type=(jax.ShapeDtypeStruct((B, D), jnp.float32),) * 4,
    mesh=_mesh(),
    scratch_types=[
        pltpu.VMEM((B // NW,), jnp.int32),
        pltpu.VMEM((B // NW, D), jnp.float32),
        pltpu.SemaphoreType.DMA,
    ],
    compiler_params=pltpu.CompilerParams(use_tc_tiling_on_sc=False),
  )


_RB = 1024


def _dinv_block(dp_ref):
    deg = jnp.sum(dp_ref[...], axis=0)[:, None] + 1.0
    return lax.rsqrt(deg)


def _store_split(o_ref, res):
    o_ref[0] = res[:, :DH]
    o_ref[1] = res[:, DH:]


def _mm_body(x_ref, w_ref, dp_ref, o_ref):
    dinv = _dinv_block(dp_ref)
    res = jnp.dot(x_ref[...], w_ref[...],
                  preferred_element_type=jnp.float32) * dinv
    _store_split(o_ref, res)


def _relu_halves(a_ref, h_ref, dinv):
    g0 = jnp.maximum((a_ref[0] + h_ref[0]) * dinv, 0.0)
    g1 = jnp.maximum((a_ref[1] + h_ref[1]) * dinv, 0.0)
    return g0, g1


def _mid_body(a_ref, h_ref, w_ref, dp_ref, o_ref):
    dinv = _dinv_block(dp_ref)
    g0, g1 = _relu_halves(a_ref, h_ref, dinv)
    res = (jnp.dot(g0, w_ref[:DH, :], preferred_element_type=jnp.float32)
           + jnp.dot(g1, w_ref[DH:, :], preferred_element_type=jnp.float32)
           ) * dinv
    _store_split(o_ref, res)


def _fin_body(a_ref, h_ref, dp_ref, o_ref):
    dinv = _dinv_block(dp_ref)
    g0, g1 = _relu_halves(a_ref, h_ref, dinv)
    o_ref[:, :DH] = g0
    o_ref[:, DH:] = g1


_row_spec = pl.BlockSpec((_RB, D), lambda i: (i, 0))
_split_spec = pl.BlockSpec((NC, _RB, DH), lambda i: (0, i, 0))
_w_spec = pl.BlockSpec((D, D), lambda i: (0, 0))
_deg_spec = pl.BlockSpec((NW, _RB), lambda i: (0, i))
_full_struct = jax.ShapeDtypeStruct((N_PAD, D), jnp.float32)
_split_struct = jax.ShapeDtypeStruct((NC, N_PAD, DH), jnp.float32)
_grid = (N_PAD // _RB,)

_tc_mm = pl.pallas_call(
    _mm_body, grid=_grid,
    in_specs=[_row_spec, _w_spec, _deg_spec],
    out_specs=_split_spec, out_shape=_split_struct)

_tc_mid = pl.pallas_call(
    _mid_body, grid=_grid,
    in_specs=[_split_spec, _split_spec, _w_spec, _deg_spec],
    out_specs=_split_spec, out_shape=_split_struct)

_tc_fin = pl.pallas_call(
    _fin_body, grid=_grid,
    in_specs=[_split_spec, _split_spec, _deg_spec],
    out_specs=_row_spec, out_shape=_full_struct)


def _prep_edges(edge_index):
    pad = NW * EP - E
    fill = jnp.full((pad,), N, jnp.int32)
    s = jnp.concatenate([edge_index[0], fill])
    d = jnp.concatenate([edge_index[1], fill])
    return (s.reshape(NS, CHB, KE), d.reshape(NS, CHB, KE),
            d.reshape(NW, CH, 128))


def kernel(entity_emb_sr, entity_emb_tg, rel_emb_sr, rel_emb_tg, W0, W1,
           edge_index_sr, edge_index_tg, sr_data, tg_data,
           sr_rel_data, tg_rel_data):
    s_sr, d_sr, d32_sr = _prep_edges(edge_index_sr)
    s_tg, d_tg, d32_tg = _prep_edges(edge_index_tg)

    deg_sr, deg_tg = _deg_kernel()(d32_sr, d32_tg)
    deg_sr = deg_sr.reshape(NW, N_PAD)
    deg_tg = deg_tg.reshape(NW, N_PAD)

    pad_rows = jnp.zeros((N_PAD - N, D), jnp.float32)
    x_sr = jnp.concatenate([entity_emb_sr, pad_rows])
    x_tg = jnp.concatenate([entity_emb_tg, pad_rows])

    h_sr = _tc_mm(x_sr, W0, deg_sr)
    h_tg = _tc_mm(x_tg, W0, deg_tg)

    agg = _agg_kernel()
    a_sr = agg(h_sr[0], h_sr[1], s_sr, d_sr)
    a_tg = agg(h_tg[0], h_tg[1], s_tg, d_tg)

    h2_sr = _tc_mid(a_sr, h_sr, W1, deg_sr)
    h2_tg = _tc_mid(a_tg, h_tg, W1, deg_tg)

    a2_sr = agg(h2_sr[0], h2_sr[1], s_sr, d_sr)
    a2_tg = agg(h2_tg[0], h2_tg[1], s_tg, d_tg)

    g_sr = _tc_fin(a2_sr, h2_sr, deg_sr)
    g_tg = _tc_fin(a2_tg, h2_tg, deg_tg)

    return _lookup_kernel()(
        g_sr, g_tg, rel_emb_sr, rel_emb_tg,
        sr_data.reshape(NW, B // NW), tg_data.reshape(NW, B // NW),
        sr_rel_data.reshape(NW, B // NW), tg_rel_data.reshape(NW, B // NW))
